# baseline (speedup 1.0000x reference)
import functools

import jax
import jax.numpy as jnp
from jax import lax
from jax.experimental import layout as jax_layout
from jax.experimental import pallas as pl
from jax.experimental.pallas import tpu as pltpu
from jax.experimental.pallas import tpu_sc as plsc

VOCAB = 1000000
EMBED = 64
BATCH = 4096
SEQ = 200

_info = plsc.get_sparse_core_info()
NC, NS = _info.num_cores, _info.num_subcores
NW = NC * NS
ROWS_PER_W = BATCH // NW
NBUF = 4
N_CHUNKS = ROWS_PER_W // NBUF


@functools.partial(
    pl.kernel,
    out_type=jax.ShapeDtypeStruct((BATCH, SEQ, EMBED), jnp.float32),
    mesh=plsc.VectorSubcoreMesh(core_axis_name="c", subcore_axis_name="s"),
    scratch_types=[
        pltpu.VMEM((ROWS_PER_W * SEQ,), jnp.int32),
        pltpu.VMEM((NBUF, SEQ, EMBED), jnp.float32),
        pltpu.SemaphoreType.DMA((NBUF,)),
        pltpu.SemaphoreType.DMA((NBUF,)),
    ],
    compiler_params=pltpu.CompilerParams(use_tc_tiling_on_sc=False),
)
def _gather_kernel(idx_hbm, table_hbm, out_hbm, idx_v, bufs, gsems, osems):
    wid = lax.axis_index("s") * NC + lax.axis_index("c")
    base = wid * ROWS_PER_W
    pltpu.sync_copy(idx_hbm.at[pl.ds(base * SEQ, ROWS_PER_W * SEQ)], idx_v)

    def gather(r, b):
        return pltpu.make_async_copy(
            table_hbm.at[idx_v.at[pl.ds(r * SEQ, SEQ)]],
            bufs.at[b],
            gsems.at[b],
        )

    def outcopy(r, b):
        return pltpu.make_async_copy(
            bufs.at[b],
            out_hbm.at[base + r],
            osems.at[b],
        )

    def chunk(c):
        for b in range(NBUF):
            r = c * NBUF + b

            @pl.when(c > 0)
            def _():
                outcopy(r - NBUF, b).wait()

            gather(r, b).start()
        for b in range(NBUF):
            r = c * NBUF + b
            gather(r, b).wait()
            outcopy(r, b).start()

    pl.loop(0, N_CHUNKS)(chunk)
    for b in range(NBUF):
        outcopy((N_CHUNKS - 1) * NBUF + b, b).wait()


def _impl(indices, wte):
    idx = indices.reshape(-1)
    out = _gather_kernel(idx, wte)
    return out.reshape(BATCH, SEQ, EMBED)


_jitted = None


def kernel(indices, wte):
    global _jitted
    if _jitted is None:
        fmt = jax_layout.Format(
            jax_layout.Layout((0, 1, 2)),
            jax.sharding.SingleDeviceSharding(jax.devices()[0]),
        )
        _jitted = jax.jit(_impl, out_shardings=fmt)
    return _jitted(indices, wte)
